# Initial kernel scaffold; baseline (speedup 1.0000x reference)
import functools

import jax
import jax.numpy as jnp
from jax import lax
from jax.experimental import pallas as pl
from jax.experimental.pallas import tpu as pltpu
from jax.experimental.pallas import tpu_sc as plsc

_B = 16384
_V = 1000000
_D = 16
_NS = 26
_ND = 13
_F = _NS + _ND

_NC = 2
_NSUB = 16
_NW = _NC * _NSUB
_BPW = _B // _NW
_CB = 128
_NCHUNK = _BPW // _CB
_IPC = _CB * _F
_G = _IPC // 128


def _sc_body(idx_hbm, emb1_hbm, emb_hbm, s_hbm, q_hbm, y1_hbm,
             idx_v, rows_v, e1_v, s_v, q_v, y1_v, sem, sem1):
    wid = lax.axis_index("s") * _NC + lax.axis_index("c")
    lanes = lax.iota(jnp.int32, 16)
    lanes39 = lanes * _F

    for c in range(_NCHUNK):
        b0 = wid * _BPW + c * _CB
        r0 = wid * (_BPW * _F // 128) + c * _G

        pltpu.sync_copy(idx_hbm.at[pl.ds(r0, _G)], idx_v)

        descs = []
        for g in range(_G):
            descs.append(pltpu.async_copy(
                emb_hbm.at[idx_v.at[g]], rows_v.at[pl.ds(g * 128, 128)], sem))
            descs.append(pltpu.async_copy(
                emb1_hbm.at[idx_v.at[g]], e1_v.at[pl.ds(g * 128, 128)], sem1))
        for d in descs:
            d.wait()

        def so_row(b, carry):
            r = b * _F
            v = rows_v[r]
            acc = v
            acc2 = v * v
            for j in range(1, _F):
                v = rows_v[r + j]
                acc = acc + v
                acc2 = acc2 + v * v
            s_v[pl.ds(b * _D, _D)] = acc
            q_v[pl.ds(b * _D, _D)] = acc2
            return carry

        lax.fori_loop(0, _CB, so_row, 0)

        for grp in range(_CB // 16):
            base = grp * 16 * _F
            acc1 = plsc.load_gather(e1_v, [lanes39 + base])
            for j in range(1, _F):
                acc1 = acc1 + plsc.load_gather(e1_v, [lanes39 + (base + j)])
            y1_v[pl.ds(grp * 16, 16)] = acc1

        pltpu.sync_copy(s_v, s_hbm.at[pl.ds(b0 * _D, _CB * _D)])
        pltpu.sync_copy(q_v, q_hbm.at[pl.ds(b0 * _D, _CB * _D)])
        pltpu.sync_copy(y1_v, y1_hbm.at[pl.ds(b0, _CB)])


def _tc_body(s_ref, q_ref, y1sc_ref, dense_ref, w1_ref, w_ref, y1_ref, y2_ref):
    df = dense_ref[...].astype(jnp.float32)
    w1 = w1_ref[...]
    y1_ref[...] = y1sc_ref[...] + jnp.sum(df * w1, axis=1, keepdims=True)
    s = s_ref[...]
    q = q_ref[...]
    for j in range(_ND):
        dj = df[:, j:j + 1]
        wj = w_ref[j:j + 1, :]
        s = s + dj * wj
        q = q + (dj * dj) * (wj * wj)
    y2_ref[...] = 0.5 * jnp.sum(s * s - q, axis=1, keepdims=True)


@jax.jit
def kernel(sparse_inputs, dense_inputs, embedding_one, embedding,
           dense_w_one, dense_w):
    idx = jnp.concatenate(
        [sparse_inputs.astype(jnp.int32), dense_inputs.astype(jnp.int32)],
        axis=1).reshape(_B * _F // 128, 128)

    mesh = plsc.VectorSubcoreMesh(
        core_axis_name="c", subcore_axis_name="s",
        num_cores=_NC, num_subcores=_NSUB)

    sc_fn = pl.kernel(
        _sc_body,
        out_type=(
            jax.ShapeDtypeStruct((_B * _D,), jnp.float32),
            jax.ShapeDtypeStruct((_B * _D,), jnp.float32),
            jax.ShapeDtypeStruct((_B,), jnp.float32),
        ),
        mesh=mesh,
        scratch_types=[
            pltpu.VMEM((_G, 128), jnp.int32),
            pltpu.VMEM((_IPC, _D), jnp.float32),
            pltpu.VMEM((_IPC,), jnp.float32),
            pltpu.VMEM((_CB * _D,), jnp.float32),
            pltpu.VMEM((_CB * _D,), jnp.float32),
            pltpu.VMEM((_CB,), jnp.float32),
            pltpu.SemaphoreType.DMA,
            pltpu.SemaphoreType.DMA,
        ],
    )

    s_flat, q_flat, y1sc = sc_fn(idx, embedding_one.reshape(_V), embedding)

    bk = 2048
    grid = _B // bk
    y1, y2 = pl.pallas_call(
        _tc_body,
        grid=(grid,),
        in_specs=[
            pl.BlockSpec((bk, _D), lambda i: (i, 0)),
            pl.BlockSpec((bk, _D), lambda i: (i, 0)),
            pl.BlockSpec((bk, 1), lambda i: (i, 0)),
            pl.BlockSpec((bk, _ND), lambda i: (i, 0)),
            pl.BlockSpec((1, _ND), lambda i: (0, 0)),
            pl.BlockSpec((_ND, _D), lambda i: (0, 0)),
        ],
        out_specs=[
            pl.BlockSpec((bk, 1), lambda i: (i, 0)),
            pl.BlockSpec((bk, 1), lambda i: (i, 0)),
        ],
        out_shape=[
            jax.ShapeDtypeStruct((_B, 1), jnp.float32),
            jax.ShapeDtypeStruct((_B, 1), jnp.float32),
        ],
    )(
        s_flat.reshape(_B, _D),
        q_flat.reshape(_B, _D),
        y1sc.reshape(_B, 1),
        dense_inputs.astype(jnp.int32),
        dense_w_one.reshape(1, _ND),
        dense_w.reshape(_ND, _D),
    )
    return (y1, y2)


# speedup vs baseline: 1.5098x; 1.5098x over previous
import functools

import jax
import jax.numpy as jnp
from jax import lax
from jax.experimental import pallas as pl
from jax.experimental.pallas import tpu as pltpu
from jax.experimental.pallas import tpu_sc as plsc

_B = 16384
_V = 1000000
_D = 16
_NS = 26
_ND = 13
_F = _NS + _ND

_NC = 2
_NSUB = 16
_NW = _NC * _NSUB
_BPW = _B // _NW
_CB = 128
_NCHUNK = _BPW // _CB
_IPC = _CB * _F
_G = _IPC // 128


def _sc_body(idx_hbm, emb1_hbm, emb_hbm, s_hbm, q_hbm, y1_hbm,
             idx_v, rows_v, e1_v, s_v, q_v, y1_v, sem, sem1):
    wid = lax.axis_index("s") * _NC + lax.axis_index("c")
    lanes = lax.iota(jnp.int32, 16)
    lanes39 = lanes * _F

    for c in range(_NCHUNK):
        b0 = wid * _BPW + c * _CB
        off = b0 * _F

        pltpu.sync_copy(idx_hbm.at[pl.ds(off, _IPC)], idx_v)

        descs = []
        for g in range(_G):
            descs.append(pltpu.async_copy(
                emb_hbm.at[idx_v.at[pl.ds(g * 128, 128)]],
                rows_v.at[pl.ds(g * 128, 128)], sem))
            descs.append(pltpu.async_copy(
                emb1_hbm.at[idx_v.at[pl.ds(g * 128, 128)]],
                e1_v.at[pl.ds(g * 128, 128)], sem1))
        for d in descs:
            d.wait()

        def so_row(b, carry):
            r = b * _F
            v = rows_v[r]
            acc = v
            acc2 = v * v
            for j in range(1, _F):
                v = rows_v[r + j]
                acc = acc + v
                acc2 = acc2 + v * v
            s_v[pl.ds(b * _D, _D)] = acc
            q_v[pl.ds(b * _D, _D)] = acc2
            return carry

        lax.fori_loop(0, _CB, so_row, 0)

        for grp in range(_CB // 16):
            base = grp * 16 * _F
            acc1 = plsc.load_gather(e1_v, [lanes39 + base])
            for j in range(1, _F):
                acc1 = acc1 + plsc.load_gather(e1_v, [lanes39 + (base + j)])
            y1_v[pl.ds(grp * 16, 16)] = acc1

        pltpu.sync_copy(s_v, s_hbm.at[pl.ds(b0 * _D, _CB * _D)])
        pltpu.sync_copy(q_v, q_hbm.at[pl.ds(b0 * _D, _CB * _D)])
        pltpu.sync_copy(y1_v, y1_hbm.at[pl.ds(b0, _CB)])


def _tc_body(s_ref, q_ref, y1sc_ref, dense_ref, w1_ref, w_ref, y1_ref, y2_ref):
    df = dense_ref[...].astype(jnp.float32)
    w1 = w1_ref[...]
    y1_ref[...] = y1sc_ref[...] + jnp.sum(df * w1, axis=1, keepdims=True)
    s = s_ref[...]
    q = q_ref[...]
    for j in range(_ND):
        dj = df[:, j:j + 1]
        wj = w_ref[j:j + 1, :]
        s = s + dj * wj
        q = q + (dj * dj) * (wj * wj)
    y2_ref[...] = 0.5 * jnp.sum(s * s - q, axis=1, keepdims=True)


@jax.jit
def kernel(sparse_inputs, dense_inputs, embedding_one, embedding,
           dense_w_one, dense_w):
    idx = jnp.concatenate(
        [sparse_inputs.astype(jnp.int32), dense_inputs.astype(jnp.int32)],
        axis=1).reshape(_B * _F)

    mesh = plsc.VectorSubcoreMesh(
        core_axis_name="c", subcore_axis_name="s",
        num_cores=_NC, num_subcores=_NSUB)

    sc_fn = pl.kernel(
        _sc_body,
        out_type=(
            jax.ShapeDtypeStruct((_B * _D,), jnp.float32),
            jax.ShapeDtypeStruct((_B * _D,), jnp.float32),
            jax.ShapeDtypeStruct((_B,), jnp.float32),
        ),
        mesh=mesh,
        scratch_types=[
            pltpu.VMEM((_IPC,), jnp.int32),
            pltpu.VMEM((_IPC, _D), jnp.float32),
            pltpu.VMEM((_IPC,), jnp.float32),
            pltpu.VMEM((_CB * _D,), jnp.float32),
            pltpu.VMEM((_CB * _D,), jnp.float32),
            pltpu.VMEM((_CB,), jnp.float32),
            pltpu.SemaphoreType.DMA,
            pltpu.SemaphoreType.DMA,
        ],
        compiler_params=pltpu.CompilerParams(
            needs_layout_passes=False, use_tc_tiling_on_sc=False),
    )

    s_flat, q_flat, y1sc = sc_fn(idx, embedding_one.reshape(_V), embedding)

    bk = 2048
    grid = _B // bk
    y1, y2 = pl.pallas_call(
        _tc_body,
        grid=(grid,),
        in_specs=[
            pl.BlockSpec((bk, _D), lambda i: (i, 0)),
            pl.BlockSpec((bk, _D), lambda i: (i, 0)),
            pl.BlockSpec((bk, 1), lambda i: (i, 0)),
            pl.BlockSpec((bk, _ND), lambda i: (i, 0)),
            pl.BlockSpec((1, _ND), lambda i: (0, 0)),
            pl.BlockSpec((_ND, _D), lambda i: (0, 0)),
        ],
        out_specs=[
            pl.BlockSpec((bk, 1), lambda i: (i, 0)),
            pl.BlockSpec((bk, 1), lambda i: (i, 0)),
        ],
        out_shape=[
            jax.ShapeDtypeStruct((_B, 1), jnp.float32),
            jax.ShapeDtypeStruct((_B, 1), jnp.float32),
        ],
    )(
        s_flat.reshape(_B, _D),
        q_flat.reshape(_B, _D),
        y1sc.reshape(_B, 1),
        dense_inputs.astype(jnp.int32),
        dense_w_one.reshape(1, _ND),
        dense_w.reshape(_ND, _D),
    )
    return (y1, y2)


# speedup vs baseline: 1.5603x; 1.0335x over previous
import jax
import jax.numpy as jnp
from jax import lax
from jax.experimental import pallas as pl
from jax.experimental.pallas import tpu as pltpu
from jax.experimental.pallas import tpu_sc as plsc

_B = 16384
_V = 1000000
_D = 16
_NS = 26
_ND = 13
_F = _NS + _ND

_NC = 2
_NSUB = 16
_NW = _NC * _NSUB
_BPW = _B // _NW
_CB = 128
_NCHUNK = _BPW // _CB
_IPC = _CB * _F


def _sc_body(idx_hbm, emb1_hbm, emb_hbm, s_hbm, q_hbm, y1_hbm,
             idx_v, rows_v, e1_v, s_v, q_v, y1_v, semi, sem, sem1):
    wid = lax.axis_index("s") * _NC + lax.axis_index("c")

    def chunk(c, carry):
        b0 = wid * _BPW + c * _CB

        idescs = []
        for j in range(_F):
            idescs.append(pltpu.async_copy(
                idx_hbm.at[pl.ds(j * _B + b0, _CB)],
                idx_v.at[pl.ds(j * _CB, _CB)], semi))
        for d in idescs:
            d.wait()

        descs = []
        for j in range(_F):
            sl = pl.ds(j * _CB, _CB)
            descs.append(pltpu.async_copy(
                emb_hbm.at[idx_v.at[sl]], rows_v.at[sl], sem))
            descs.append(pltpu.async_copy(
                emb1_hbm.at[idx_v.at[sl]], e1_v.at[sl], sem1))
        for d in descs:
            d.wait()

        def so_row(b, carry2):
            v = rows_v[b]
            acc = v
            acc2 = v * v
            for j in range(1, _F):
                v = rows_v[j * _CB + b]
                acc = acc + v
                acc2 = acc2 + v * v
            s_v[pl.ds(b * _D, _D)] = acc
            q_v[pl.ds(b * _D, _D)] = acc2
            return carry2

        lax.fori_loop(0, _CB, so_row, 0, unroll=2)

        def fo_grp(g, carry2):
            acc1 = e1_v[pl.ds(g * 16, 16)]
            for j in range(1, _F):
                acc1 = acc1 + e1_v[pl.ds(j * _CB + g * 16, 16)]
            y1_v[pl.ds(g * 16, 16)] = acc1
            return carry2

        lax.fori_loop(0, _CB // 16, fo_grp, 0)

        pltpu.sync_copy(s_v, s_hbm.at[pl.ds(b0 * _D, _CB * _D)])
        pltpu.sync_copy(q_v, q_hbm.at[pl.ds(b0 * _D, _CB * _D)])
        pltpu.sync_copy(y1_v, y1_hbm.at[pl.ds(b0, _CB)])
        return carry

    lax.fori_loop(0, _NCHUNK, chunk, 0)


def _tc_body(s_ref, q_ref, y1sc_ref, dense_ref, w1_ref, w_ref, y1_ref, y2_ref):
    df = dense_ref[...].astype(jnp.float32)
    w1 = w1_ref[...]
    y1_ref[...] = y1sc_ref[...] + jnp.sum(df * w1, axis=1, keepdims=True)
    s = s_ref[...]
    q = q_ref[...]
    for j in range(_ND):
        dj = df[:, j:j + 1]
        wj = w_ref[j:j + 1, :]
        s = s + dj * wj
        q = q + (dj * dj) * (wj * wj)
    y2_ref[...] = 0.5 * jnp.sum(s * s - q, axis=1, keepdims=True)


@jax.jit
def kernel(sparse_inputs, dense_inputs, embedding_one, embedding,
           dense_w_one, dense_w):
    idx = jnp.concatenate(
        [jnp.transpose(sparse_inputs.astype(jnp.int32)),
         jnp.transpose(dense_inputs.astype(jnp.int32))], axis=0).reshape(-1)

    mesh = plsc.VectorSubcoreMesh(
        core_axis_name="c", subcore_axis_name="s",
        num_cores=_NC, num_subcores=_NSUB)

    sc_fn = pl.kernel(
        _sc_body,
        out_type=(
            jax.ShapeDtypeStruct((_B * _D,), jnp.float32),
            jax.ShapeDtypeStruct((_B * _D,), jnp.float32),
            jax.ShapeDtypeStruct((_B,), jnp.float32),
        ),
        mesh=mesh,
        scratch_types=[
            pltpu.VMEM((_IPC,), jnp.int32),
            pltpu.VMEM((_IPC, _D), jnp.float32),
            pltpu.VMEM((_IPC,), jnp.float32),
            pltpu.VMEM((_CB * _D,), jnp.float32),
            pltpu.VMEM((_CB * _D,), jnp.float32),
            pltpu.VMEM((_CB,), jnp.float32),
            pltpu.SemaphoreType.DMA,
            pltpu.SemaphoreType.DMA,
            pltpu.SemaphoreType.DMA,
        ],
        compiler_params=pltpu.CompilerParams(
            needs_layout_passes=False, use_tc_tiling_on_sc=False),
    )

    s_flat, q_flat, y1sc = sc_fn(idx, embedding_one.reshape(_V), embedding)

    bk = 2048
    grid = _B // bk
    y1, y2 = pl.pallas_call(
        _tc_body,
        grid=(grid,),
        in_specs=[
            pl.BlockSpec((bk, _D), lambda i: (i, 0)),
            pl.BlockSpec((bk, _D), lambda i: (i, 0)),
            pl.BlockSpec((bk, 1), lambda i: (i, 0)),
            pl.BlockSpec((bk, _ND), lambda i: (i, 0)),
            pl.BlockSpec((1, _ND), lambda i: (0, 0)),
            pl.BlockSpec((_ND, _D), lambda i: (0, 0)),
        ],
        out_specs=[
            pl.BlockSpec((bk, 1), lambda i: (i, 0)),
            pl.BlockSpec((bk, 1), lambda i: (i, 0)),
        ],
        out_shape=[
            jax.ShapeDtypeStruct((_B, 1), jnp.float32),
            jax.ShapeDtypeStruct((_B, 1), jnp.float32),
        ],
    )(
        s_flat.reshape(_B, _D),
        q_flat.reshape(_B, _D),
        y1sc.reshape(_B, 1),
        dense_inputs.astype(jnp.int32),
        dense_w_one.reshape(1, _ND),
        dense_w.reshape(_ND, _D),
    )
    return (y1, y2)


# speedup vs baseline: 1.8292x; 1.1723x over previous
import jax
import jax.numpy as jnp
from jax import lax
from jax.experimental import pallas as pl
from jax.experimental.pallas import tpu as pltpu
from jax.experimental.pallas import tpu_sc as plsc

_B = 16384
_V = 1000000
_D = 16
_NS = 26
_ND = 13
_F = _NS + _ND

_NC = 2
_NSUB = 16
_NW = _NC * _NSUB
_BPW = _B // _NW
_CB = 128
_NCHUNK = _BPW // _CB
_IPC = _CB * _F


def _sc_body(idx_hbm, emb1_hbm, emb_hbm, w1_hbm, w_hbm, y1_hbm, y2_hbm,
             idx_v, rows_v, e1_v, df_v, dfb_v, t_v, y1_v, y2_v, w1_v, w_v,
             semi, sem, sem1):
    wid = lax.axis_index("s") * _NC + lax.axis_index("c")
    lanes = lax.iota(jnp.int32, 16)
    lanes16 = lanes * _D

    pltpu.sync_copy(w1_hbm, w1_v)
    pltpu.sync_copy(w_hbm, w_v)
    w_rows = [w_v[pl.ds(j * _D, _D)] for j in range(_ND)]
    w2_rows = [w * w for w in w_rows]
    w1_vec = w1_v[...]

    def chunk(c, carry):
        b0 = wid * _BPW + c * _CB

        idescs = []
        for j in range(_F):
            idescs.append(pltpu.async_copy(
                idx_hbm.at[pl.ds(j * _B + b0, _CB)],
                idx_v.at[pl.ds(j * _CB, _CB)], semi))
        for d in idescs:
            d.wait()

        descs = []
        for j in range(_F):
            sl = pl.ds(j * _CB, _CB)
            descs.append(pltpu.async_copy(
                emb_hbm.at[idx_v.at[sl]], rows_v.at[sl], sem))
            descs.append(pltpu.async_copy(
                emb1_hbm.at[idx_v.at[sl]], e1_v.at[sl], sem1))

        def conv_grp(g, carry2):
            for jd in range(_ND):
                sl_i = pl.ds((_NS + jd) * _CB + g * 16, 16)
                sl_o = pl.ds(jd * _CB + g * 16, 16)
                cvec = idx_v[sl_i].astype(jnp.float32)
                df_v[sl_o] = cvec
                plsc.store_scatter(dfb_v, [lanes16 + (g * 256 + jd)], cvec)
            return carry2

        lax.fori_loop(0, _CB // 16, conv_grp, 0)

        for d in descs:
            d.wait()

        def so_row(b, carry2):
            v = rows_v[b]
            acc = v
            acc2 = v * v
            for j in range(1, _F):
                v = rows_v[j * _CB + b]
                acc = acc + v
                acc2 = acc2 + v * v
            dfv = dfb_v[pl.ds(b * _D, _D)]
            for jd in range(_ND):
                dfs = dfv[jd]
                acc = acc + dfs * w_rows[jd]
                acc2 = acc2 + (dfs * dfs) * w2_rows[jd]
            t_v[pl.ds(b * _D, _D)] = acc * acc - acc2
            return carry2

        lax.fori_loop(0, _CB, so_row, 0, unroll=2)

        def fo_grp(g, carry2):
            acc1 = e1_v[pl.ds(g * 16, 16)]
            for j in range(1, _F):
                acc1 = acc1 + e1_v[pl.ds(j * _CB + g * 16, 16)]
            for jd in range(_ND):
                acc1 = acc1 + df_v[pl.ds(jd * _CB + g * 16, 16)] * w1_vec[jd]
            y1_v[pl.ds(g * 16, 16)] = acc1

            tl = lanes16 + g * (16 * _D)
            acc2 = plsc.load_gather(t_v, [tl])
            for d in range(1, _D):
                acc2 = acc2 + plsc.load_gather(t_v, [tl + d])
            y2_v[pl.ds(g * 16, 16)] = 0.5 * acc2
            return carry2

        lax.fori_loop(0, _CB // 16, fo_grp, 0)

        pltpu.sync_copy(y1_v, y1_hbm.at[pl.ds(b0, _CB)])
        pltpu.sync_copy(y2_v, y2_hbm.at[pl.ds(b0, _CB)])
        return carry

    lax.fori_loop(0, _NCHUNK, chunk, 0)


@jax.jit
def kernel(sparse_inputs, dense_inputs, embedding_one, embedding,
           dense_w_one, dense_w):
    idx = jnp.concatenate(
        [jnp.transpose(sparse_inputs.astype(jnp.int32)),
         jnp.transpose(dense_inputs.astype(jnp.int32))], axis=0).reshape(-1)
    w1p = jnp.pad(dense_w_one.astype(jnp.float32), (0, 3))
    wf = dense_w.astype(jnp.float32).reshape(_ND * _D)

    mesh = plsc.VectorSubcoreMesh(
        core_axis_name="c", subcore_axis_name="s",
        num_cores=_NC, num_subcores=_NSUB)

    sc_fn = pl.kernel(
        _sc_body,
        out_type=(
            jax.ShapeDtypeStruct((_B,), jnp.float32),
            jax.ShapeDtypeStruct((_B,), jnp.float32),
        ),
        mesh=mesh,
        scratch_types=[
            pltpu.VMEM((_IPC,), jnp.int32),
            pltpu.VMEM((_IPC, _D), jnp.float32),
            pltpu.VMEM((_IPC,), jnp.float32),
            pltpu.VMEM((_ND * _CB,), jnp.float32),
            pltpu.VMEM((_CB * _D,), jnp.float32),
            pltpu.VMEM((_CB * _D,), jnp.float32),
            pltpu.VMEM((_CB,), jnp.float32),
            pltpu.VMEM((_CB,), jnp.float32),
            pltpu.VMEM((16,), jnp.float32),
            pltpu.VMEM((_ND * _D,), jnp.float32),
            pltpu.SemaphoreType.DMA,
            pltpu.SemaphoreType.DMA,
            pltpu.SemaphoreType.DMA,
        ],
        compiler_params=pltpu.CompilerParams(
            needs_layout_passes=False, use_tc_tiling_on_sc=False),
    )

    y1, y2 = sc_fn(idx, embedding_one.reshape(_V), embedding, w1p, wf)
    return (y1.reshape(_B, 1), y2.reshape(_B, 1))
